# baseline (speedup 1.0000x reference)
import jax
import jax.numpy as jnp
from jax import lax
from jax.experimental import pallas as pl
from jax.experimental.pallas import tpu as pltpu

B, FEAT, E, L, V, T = 128, 512, 8, 256, 50257, 16
BE = B * E
BT = B * T
VT = 4096
NT = (V + VT - 1) // VT
VQ = 512
CH = (BT + NT - 2) // (NT - 1)
NMC = (V - (NT - 1) * VT + (VT // 8) - 1) // (VT // 8)


def _mos_body(tgt_ref, feat_ref, wexp_ref, bexp_ref, wpri_ref, bpri_ref,
              wl_ref, bl_ref, wlab_hbm, tcol_ref, dbp_ref,
              o_ref, eh_s, eh8_s, pri_s, s_s, wg_s, sem):
    g = pl.program_id(0)

    lo = jnp.minimum(g * CH, BT)
    hi = jnp.minimum(lo + CH, BT)

    def issue(i, c):
        idx = tgt_ref[i]
        pltpu.make_async_copy(wlab_hbm.at[pl.ds(idx, 1), :],
                              wg_s.at[pl.ds(i, 1), :], sem).start()
        return c

    lax.fori_loop(lo, hi, issue, 0)

    @pl.when(g == 0)
    def _():
        f = feat_ref[...].astype(jnp.bfloat16)
        we = wexp_ref[...].astype(jnp.bfloat16)
        eh = lax.dot_general(f, we, (((1,), (1,)), ((), ())),
                             preferred_element_type=jnp.float32)
        eh = (eh + bexp_ref[...]).astype(jnp.bfloat16)
        for e in range(E):
            eht = jnp.transpose(eh[:, e * L:(e + 1) * L])
            eh_s[:, e * B:(e + 1) * B] = eht
            eh8_s[:, e * B:(e + 1) * B] = eht.astype(jnp.float8_e4m3fn)

        wp = wpri_ref[...].astype(jnp.bfloat16)
        pr = lax.dot_general(wp, f, (((1,), (1,)), ((), ())),
                             preferred_element_type=jnp.float32)
        pr = pr + bpri_ref[...]
        pe = jnp.exp(pr)
        pri_s[...] = pe / jnp.sum(pe, axis=0, keepdims=True)

        s_s[...] = jnp.zeros_like(s_s)

    ehb = eh_s[...]
    bl = bl_ref[...].astype(jnp.bfloat16)
    CW = VT // 8

    eh8 = eh8_s[...]

    def chunk(c, masked):
        wc = wl_ref[c * CW:(c + 1) * CW, :].astype(jnp.float8_e4m3fn)
        lgc = lax.dot_general(eh8, wc, (((0,), (1,)), ((), ())),
                              preferred_element_type=jnp.float32
                              ).astype(jnp.bfloat16)
        y = lgc + bl[:, c * CW:(c + 1) * CW]
        if masked:
            vidx = (g * VT + c * CW +
                    lax.broadcasted_iota(jnp.int32, (1, CW), 1))
            y = jnp.where(vidx < V, y, jnp.bfloat16(-1e30))
        x = jnp.exp(y)
        ps = x[:, 0:128]
        for j in range(1, CW // 128):
            ps = ps + x[:, j * 128:(j + 1) * 128]
        return ps.astype(jnp.float32)

    def accum(masked, nch):
        ps = [chunk(c, masked) for c in range(nch)]
        while len(ps) > 1:
            ps = [a + b for a, b in zip(ps[::2], ps[1::2])] + (
                [ps[-1]] if len(ps) % 2 else [])
        s_s[...] += ps[0]

    @pl.when(g < NT - 1)
    def _():
        accum(False, VT // 512)

    @pl.when(g == NT - 1)
    def _():
        accum(True, NMC)

        pltpu.make_async_copy(wg_s, wg_s, sem).wait()

        wgb = wg_s[...].astype(jnp.bfloat16)
        tl = lax.dot_general(wgb, ehb, (((1,), (0,)), ((), ())),
                             preferred_element_type=jnp.float32)

        st = jnp.transpose(s_s[...])
        s_row = jnp.sum(st, axis=0, keepdims=True)

        tlb = tl.astype(jnp.bfloat16)
        acc = jnp.zeros((BT, 128), jnp.float32)
        for e in range(E):
            w_e = (pri_s[e:e + 1, :] /
                   s_row[:, e * B:(e + 1) * B]).astype(jnp.bfloat16)
            xe = jnp.exp(tlb[:, e * B:(e + 1) * B]) * w_e
            acc = acc + xe.astype(jnp.float32)
        rbm = lax.broadcasted_iota(jnp.int32, (BT, 1), 0) >> 4
        cbm = lax.broadcasted_iota(jnp.int32, (1, B), 1)
        pm = jnp.sum(jnp.where(rbm == cbm, acc, 0.0), axis=-1,
                     keepdims=True)

        tc = tcol_ref[...]
        q = tc >> 7
        r = tc & 127
        iot = lax.broadcasted_iota(jnp.int32, (1, VQ), 1)
        oh = jnp.where(q == iot, 1.0, 0.0).astype(jnp.bfloat16)
        gg = lax.dot_general(oh, dbp_ref[...].astype(jnp.bfloat16),
                             (((1,), (0,)), ((), ())),
                             preferred_element_type=jnp.float32)
        dpv = jnp.take_along_axis(gg[:, 0:128], r, axis=1)
        blv = jnp.take_along_axis(gg[:, 128:256], r, axis=1)

        lp = jnp.log(pm) + blv
        ratio = 1.0 - dpv
        num = (-lp) * ratio
        nums = jnp.sum(num.reshape(B, T, 1), axis=1)
        dens = jnp.sum(ratio.reshape(B, T, 1), axis=1)
        ps = nums / dens
        o_ref[...] = (jnp.sum(ps) / (B + 1e-5)).reshape(1, 1)


def kernel(features, W_exp, b_exp, W_lab, b_lab, W_pri, b_pri,
           discard_probs, targets):
    bexp2 = b_exp.reshape(1, E * L)
    bpri_c = b_pri.reshape(E, 1)
    bl2 = b_lab.reshape(1, V)

    npad = VQ * 128 - V
    dp_p = jnp.pad(discard_probs, (0, npad)).reshape(VQ, 128)
    bl_p = jnp.pad(b_lab, (0, npad)).reshape(VQ, 128)
    dbp = jnp.concatenate([dp_p, bl_p], axis=1)

    tflat = targets.astype(jnp.int32).reshape(BT)
    tcol = targets.astype(jnp.int32).reshape(BT, 1)

    loss = pl.pallas_call(
        _mos_body,
        grid_spec=pltpu.PrefetchScalarGridSpec(
            num_scalar_prefetch=1,
            grid=(NT,),
            in_specs=[
                pl.BlockSpec((B, FEAT), lambda i, s: (0, 0)),
                pl.BlockSpec((E * L, FEAT), lambda i, s: (0, 0)),
                pl.BlockSpec((1, E * L), lambda i, s: (0, 0)),
                pl.BlockSpec((E, FEAT), lambda i, s: (0, 0)),
                pl.BlockSpec((E, 1), lambda i, s: (0, 0)),
                pl.BlockSpec((VT, L), lambda i, s: (i, 0)),
                pl.BlockSpec((1, VT), lambda i, s: (0, i)),
                pl.BlockSpec(memory_space=pl.ANY),
                pl.BlockSpec((BT, 1), lambda i, s: (0, 0)),
                pl.BlockSpec((VQ, 256), lambda i, s: (0, 0)),
            ],
            out_specs=pl.BlockSpec((1, 1), lambda i, s: (0, 0)),
            scratch_shapes=[
                pltpu.VMEM((L, BE), jnp.bfloat16),
                pltpu.VMEM((L, BE), jnp.float8_e4m3fn),
                pltpu.VMEM((E, B), jnp.float32),
                pltpu.VMEM((BE, 128), jnp.float32),
                pltpu.VMEM((BT, L), jnp.float32),
                pltpu.SemaphoreType.DMA,
            ],
        ),
        out_shape=jax.ShapeDtypeStruct((1, 1), jnp.float32),
        compiler_params=pltpu.CompilerParams(
            dimension_semantics=("arbitrary",),
            vmem_limit_bytes=56 * 1024 * 1024,
            disable_bounds_checks=True),
        name="mos_loss_fused",
    )(tflat, features, W_exp, bexp2, W_pri, bpri_c, W_lab, bl2, W_lab,
      tcol, dbp)

    return loss.reshape(1)


# speedup vs baseline: 1.0401x; 1.0401x over previous
import jax
import jax.numpy as jnp
from jax import lax
from jax.experimental import pallas as pl
from jax.experimental.pallas import tpu as pltpu

B, FEAT, E, L, V, T = 128, 512, 8, 256, 50257, 16
BE = B * E
BT = B * T
VT = 8192
NT = (V + VT - 1) // VT
VQ = 512
CH = (BT + NT - 2) // (NT - 1)
NMC = (V - (NT - 1) * VT + 511) // 512


def _mos_body(tgt_ref, feat_ref, wexp_ref, bexp_ref, wpri_ref, bpri_ref,
              wl_ref, bl_ref, wlab_hbm, tcol_ref, dbp_ref,
              o_ref, eh_s, pri_s, s_s, wg_s, sem):
    g = pl.program_id(0)

    lo = jnp.minimum(g * CH, BT)
    hi = jnp.minimum(lo + CH, BT)

    def issue(i, c):
        idx = tgt_ref[i]
        pltpu.make_async_copy(wlab_hbm.at[pl.ds(idx, 1), :],
                              wg_s.at[pl.ds(i, 1), :], sem).start()
        return c

    lax.fori_loop(lo, hi, issue, 0)

    @pl.when(g == 0)
    def _():
        f = feat_ref[...].astype(jnp.bfloat16)
        we = wexp_ref[...].astype(jnp.bfloat16)
        eh = lax.dot_general(f, we, (((1,), (1,)), ((), ())),
                             preferred_element_type=jnp.float32)
        eh = (eh + bexp_ref[...]).astype(jnp.bfloat16)
        for e in range(E):
            eh_s[:, e * B:(e + 1) * B] = jnp.transpose(
                eh[:, e * L:(e + 1) * L])

        wp = wpri_ref[...].astype(jnp.bfloat16)
        pr = lax.dot_general(wp, f, (((1,), (1,)), ((), ())),
                             preferred_element_type=jnp.float32)
        pr = pr + bpri_ref[...]
        pe = jnp.exp(pr)
        pri_s[...] = pe / jnp.sum(pe, axis=0, keepdims=True)

        s_s[...] = jnp.zeros_like(s_s)

    ehb = eh_s[...]
    bl = bl_ref[...].astype(jnp.bfloat16)
    CW = 512

    def chunk(c, masked):
        wc = wl_ref[c * CW:(c + 1) * CW, :].astype(jnp.bfloat16)
        lgc = lax.dot_general(ehb, wc, (((0,), (1,)), ((), ())),
                              preferred_element_type=jnp.float32
                              ).astype(jnp.bfloat16)
        y = lgc + bl[:, c * CW:(c + 1) * CW]
        if masked:
            vidx = (g * VT + c * CW +
                    lax.broadcasted_iota(jnp.int32, (1, CW), 1))
            y = jnp.where(vidx < V, y, jnp.bfloat16(-1e30))
        x = jnp.exp(y)
        ps = x[:, 0:128]
        for j in range(1, CW // 128):
            ps = ps + x[:, j * 128:(j + 1) * 128]
        return ps.astype(jnp.float32)

    def accum(masked, nch):
        ps = [chunk(c, masked) for c in range(nch)]
        while len(ps) > 1:
            ps = [a + b for a, b in zip(ps[::2], ps[1::2])] + (
                [ps[-1]] if len(ps) % 2 else [])
        s_s[...] += ps[0]

    @pl.when(g < NT - 1)
    def _():
        accum(False, VT // 512)

    @pl.when(g == NT - 1)
    def _():
        accum(True, NMC)

        pltpu.make_async_copy(wg_s, wg_s, sem).wait()

        wgb = wg_s[...].astype(jnp.bfloat16)
        tl = lax.dot_general(wgb, ehb, (((1,), (0,)), ((), ())),
                             preferred_element_type=jnp.float32)

        st = jnp.transpose(s_s[...])
        s_row = jnp.sum(st, axis=0, keepdims=True)

        tlb = tl.astype(jnp.bfloat16)
        acc = jnp.zeros((BT, 128), jnp.float32)
        for e in range(E):
            w_e = (pri_s[e:e + 1, :] /
                   s_row[:, e * B:(e + 1) * B]).astype(jnp.bfloat16)
            xe = jnp.exp(tlb[:, e * B:(e + 1) * B]) * w_e
            acc = acc + xe.astype(jnp.float32)
        rbm = lax.broadcasted_iota(jnp.int32, (BT, 1), 0) >> 4
        cbm = lax.broadcasted_iota(jnp.int32, (1, B), 1)
        pm = jnp.sum(jnp.where(rbm == cbm, acc, 0.0), axis=-1,
                     keepdims=True)

        tc = tcol_ref[...]
        q = tc >> 7
        r = tc & 127
        iot = lax.broadcasted_iota(jnp.int32, (1, VQ), 1)
        oh = jnp.where(q == iot, 1.0, 0.0).astype(jnp.bfloat16)
        gg = lax.dot_general(oh, dbp_ref[...].astype(jnp.bfloat16),
                             (((1,), (0,)), ((), ())),
                             preferred_element_type=jnp.float32)
        dpv = jnp.take_along_axis(gg[:, 0:128], r, axis=1)
        blv = jnp.take_along_axis(gg[:, 128:256], r, axis=1)

        lp = jnp.log(pm) + blv
        ratio = 1.0 - dpv
        num = (-lp) * ratio
        nums = jnp.sum(num.reshape(B, T, 1), axis=1)
        dens = jnp.sum(ratio.reshape(B, T, 1), axis=1)
        ps = nums / dens
        o_ref[...] = (jnp.sum(ps) / (B + 1e-5)).reshape(1, 1)


def kernel(features, W_exp, b_exp, W_lab, b_lab, W_pri, b_pri,
           discard_probs, targets):
    bexp2 = b_exp.reshape(1, E * L)
    bpri_c = b_pri.reshape(E, 1)
    bl2 = b_lab.reshape(1, V)

    npad = VQ * 128 - V
    dp_p = jnp.pad(discard_probs, (0, npad)).reshape(VQ, 128)
    bl_p = jnp.pad(b_lab, (0, npad)).reshape(VQ, 128)
    dbp = jnp.concatenate([dp_p, bl_p], axis=1)

    tflat = targets.astype(jnp.int32).reshape(BT)
    tcol = targets.astype(jnp.int32).reshape(BT, 1)

    loss = pl.pallas_call(
        _mos_body,
        grid_spec=pltpu.PrefetchScalarGridSpec(
            num_scalar_prefetch=1,
            grid=(NT,),
            in_specs=[
                pl.BlockSpec((B, FEAT), lambda i, s: (0, 0)),
                pl.BlockSpec((E * L, FEAT), lambda i, s: (0, 0)),
                pl.BlockSpec((1, E * L), lambda i, s: (0, 0)),
                pl.BlockSpec((E, FEAT), lambda i, s: (0, 0)),
                pl.BlockSpec((E, 1), lambda i, s: (0, 0)),
                pl.BlockSpec((VT, L), lambda i, s: (i, 0)),
                pl.BlockSpec((1, VT), lambda i, s: (0, i)),
                pl.BlockSpec(memory_space=pl.ANY),
                pl.BlockSpec((BT, 1), lambda i, s: (0, 0)),
                pl.BlockSpec((VQ, 256), lambda i, s: (0, 0)),
            ],
            out_specs=pl.BlockSpec((1, 1), lambda i, s: (0, 0)),
            scratch_shapes=[
                pltpu.VMEM((L, BE), jnp.bfloat16),
                pltpu.VMEM((E, B), jnp.float32),
                pltpu.VMEM((BE, 128), jnp.float32),
                pltpu.VMEM((BT, L), jnp.float32),
                pltpu.SemaphoreType.DMA,
            ],
        ),
        out_shape=jax.ShapeDtypeStruct((1, 1), jnp.float32),
        compiler_params=pltpu.CompilerParams(
            dimension_semantics=("arbitrary",),
            vmem_limit_bytes=56 * 1024 * 1024,
            disable_bounds_checks=True),
        name="mos_loss_fused",
    )(tflat, features, W_exp, bexp2, W_pri, bpri_c, W_lab, bl2, W_lab,
      tcol, dbp)

    return loss.reshape(1)


# speedup vs baseline: 1.0607x; 1.0198x over previous
import jax
import jax.numpy as jnp
from jax import lax
from jax.experimental import pallas as pl
from jax.experimental.pallas import tpu as pltpu

B, FEAT, E, L, V, T = 128, 512, 8, 256, 50257, 16
BE = B * E
BT = B * T
VT = 8192
NT = (V + VT - 1) // VT
VQ = 512
CH = (BT + NT - 2) // (NT - 1)
NMC = (V - (NT - 1) * VT + 255) // 256


def _mos_body(tgt_ref, feat_ref, wexp_ref, bexp_ref, wpri_ref, bpri_ref,
              wl_ref, bl_ref, wlab_hbm, tcol_ref, dbp_ref,
              o_ref, eh_s, pri_s, s_s, wg_s, sem):
    g = pl.program_id(0)

    lo = jnp.minimum(g * CH, BT)
    hi = jnp.minimum(lo + CH, BT)

    def issue(i, c):
        idx = tgt_ref[i]
        pltpu.make_async_copy(wlab_hbm.at[pl.ds(idx, 1), :],
                              wg_s.at[pl.ds(i, 1), :], sem).start()
        return c

    lax.fori_loop(lo, hi, issue, 0)

    @pl.when(g == 0)
    def _():
        f = feat_ref[...].astype(jnp.bfloat16)
        we = wexp_ref[...].astype(jnp.bfloat16)
        eh = lax.dot_general(f, we, (((1,), (1,)), ((), ())),
                             preferred_element_type=jnp.float32)
        eh = (eh + bexp_ref[...]).astype(jnp.bfloat16)
        for e in range(E):
            eh_s[:, e * B:(e + 1) * B] = jnp.transpose(
                eh[:, e * L:(e + 1) * L])

        wp = wpri_ref[...].astype(jnp.bfloat16)
        pr = lax.dot_general(wp, f, (((1,), (1,)), ((), ())),
                             preferred_element_type=jnp.float32)
        pr = pr + bpri_ref[...]
        pe = jnp.exp(pr)
        pri_s[...] = pe / jnp.sum(pe, axis=0, keepdims=True)

        s_s[...] = jnp.zeros_like(s_s)

    ehb = eh_s[...]
    bl = bl_ref[...].astype(jnp.bfloat16)
    CW = 256

    def chunk(c, masked):
        wc = wl_ref[c * CW:(c + 1) * CW, :].astype(jnp.bfloat16)
        lgc = lax.dot_general(ehb, wc, (((0,), (1,)), ((), ())),
                              preferred_element_type=jnp.float32
                              ).astype(jnp.bfloat16)
        y = lgc + bl[:, c * CW:(c + 1) * CW]
        if masked:
            vidx = (g * VT + c * CW +
                    lax.broadcasted_iota(jnp.int32, (1, CW), 1))
            y = jnp.where(vidx < V, y, jnp.bfloat16(-1e30))
        x = jnp.exp(y)
        ps = x[:, 0:128]
        for j in range(1, CW // 128):
            ps = ps + x[:, j * 128:(j + 1) * 128]
        return ps.astype(jnp.float32)

    def accum(masked, nch):
        ps = [chunk(c, masked) for c in range(nch)]
        while len(ps) > 1:
            ps = [a + b for a, b in zip(ps[::2], ps[1::2])] + (
                [ps[-1]] if len(ps) % 2 else [])
        s_s[...] += ps[0]

    @pl.when(g < NT - 1)
    def _():
        accum(False, VT // 256)

    @pl.when(g == NT - 1)
    def _():
        accum(True, NMC)

        pltpu.make_async_copy(wg_s, wg_s, sem).wait()

        wgb = wg_s[...].astype(jnp.bfloat16)
        tl = lax.dot_general(wgb, ehb, (((1,), (0,)), ((), ())),
                             preferred_element_type=jnp.float32)

        st = jnp.transpose(s_s[...])
        s_row = jnp.sum(st, axis=0, keepdims=True)

        tlb = tl.astype(jnp.bfloat16)
        acc = jnp.zeros((BT, 128), jnp.float32)
        for e in range(E):
            w_e = (pri_s[e:e + 1, :] /
                   s_row[:, e * B:(e + 1) * B]).astype(jnp.bfloat16)
            xe = jnp.exp(tlb[:, e * B:(e + 1) * B]) * w_e
            acc = acc + xe.astype(jnp.float32)
        rbm = lax.broadcasted_iota(jnp.int32, (BT, 1), 0) >> 4
        cbm = lax.broadcasted_iota(jnp.int32, (1, B), 1)
        pm = jnp.sum(jnp.where(rbm == cbm, acc, 0.0), axis=-1,
                     keepdims=True)

        tc = tcol_ref[...]
        q = tc >> 7
        r = tc & 127
        iot = lax.broadcasted_iota(jnp.int32, (1, VQ), 1)
        oh = jnp.where(q == iot, 1.0, 0.0).astype(jnp.bfloat16)
        gg = lax.dot_general(oh, dbp_ref[...].astype(jnp.bfloat16),
                             (((1,), (0,)), ((), ())),
                             preferred_element_type=jnp.float32)
        dpv = jnp.take_along_axis(gg[:, 0:128], r, axis=1)
        blv = jnp.take_along_axis(gg[:, 128:256], r, axis=1)

        lp = jnp.log(pm) + blv
        ratio = 1.0 - dpv
        num = (-lp) * ratio
        nums = jnp.sum(num.reshape(B, T, 1), axis=1)
        dens = jnp.sum(ratio.reshape(B, T, 1), axis=1)
        ps = nums / dens
        o_ref[...] = (jnp.sum(ps) / (B + 1e-5)).reshape(1, 1)


def kernel(features, W_exp, b_exp, W_lab, b_lab, W_pri, b_pri,
           discard_probs, targets):
    bexp2 = b_exp.reshape(1, E * L)
    bpri_c = b_pri.reshape(E, 1)
    bl2 = b_lab.reshape(1, V)

    npad = VQ * 128 - V
    dp_p = jnp.pad(discard_probs, (0, npad)).reshape(VQ, 128)
    bl_p = jnp.pad(b_lab, (0, npad)).reshape(VQ, 128)
    dbp = jnp.concatenate([dp_p, bl_p], axis=1)

    tflat = targets.astype(jnp.int32).reshape(BT)
    tcol = targets.astype(jnp.int32).reshape(BT, 1)

    loss = pl.pallas_call(
        _mos_body,
        grid_spec=pltpu.PrefetchScalarGridSpec(
            num_scalar_prefetch=1,
            grid=(NT,),
            in_specs=[
                pl.BlockSpec((B, FEAT), lambda i, s: (0, 0)),
                pl.BlockSpec((E * L, FEAT), lambda i, s: (0, 0)),
                pl.BlockSpec((1, E * L), lambda i, s: (0, 0)),
                pl.BlockSpec((E, FEAT), lambda i, s: (0, 0)),
                pl.BlockSpec((E, 1), lambda i, s: (0, 0)),
                pl.BlockSpec((VT, L), lambda i, s: (i, 0)),
                pl.BlockSpec((1, VT), lambda i, s: (0, i)),
                pl.BlockSpec(memory_space=pl.ANY),
                pl.BlockSpec((BT, 1), lambda i, s: (0, 0)),
                pl.BlockSpec((VQ, 256), lambda i, s: (0, 0)),
            ],
            out_specs=pl.BlockSpec((1, 1), lambda i, s: (0, 0)),
            scratch_shapes=[
                pltpu.VMEM((L, BE), jnp.bfloat16),
                pltpu.VMEM((E, B), jnp.float32),
                pltpu.VMEM((BE, 128), jnp.float32),
                pltpu.VMEM((BT, L), jnp.float32),
                pltpu.SemaphoreType.DMA,
            ],
        ),
        out_shape=jax.ShapeDtypeStruct((1, 1), jnp.float32),
        compiler_params=pltpu.CompilerParams(
            dimension_semantics=("arbitrary",),
            vmem_limit_bytes=56 * 1024 * 1024,
            disable_bounds_checks=True),
        name="mos_loss_fused",
    )(tflat, features, W_exp, bexp2, W_pri, bpri_c, W_lab, bl2, W_lab,
      tcol, dbp)

    return loss.reshape(1)


# speedup vs baseline: 1.0643x; 1.0034x over previous
import jax
import jax.numpy as jnp
from jax import lax
from jax.experimental import pallas as pl
from jax.experimental.pallas import tpu as pltpu

B, FEAT, E, L, V, T = 128, 512, 8, 256, 50257, 16
BE = B * E
BT = B * T
VT = 8192
NT = (V + VT - 1) // VT
VQ = 512
CH = (BT + NT - 2) // (NT - 1)
NMC = (V - (NT - 1) * VT + 255) // 256


def _mos_body(tgt_ref, feat_ref, wexp_ref, bexp_ref, wpri_ref, bpri_ref,
              wl_ref, bl_ref, wlab_hbm, tcol_ref, dbp_ref,
              o_ref, eh_s, pri_s, s_s, wg_s, rat_s, blv_s, sem):
    g = pl.program_id(0)

    lo = jnp.minimum(g * CH, BT)
    hi = jnp.minimum(lo + CH, BT)

    def issue(i, c):
        idx = tgt_ref[i]
        pltpu.make_async_copy(wlab_hbm.at[pl.ds(idx, 1), :],
                              wg_s.at[pl.ds(i, 1), :], sem).start()
        return c

    lax.fori_loop(lo, hi, issue, 0)

    @pl.when(g == 0)
    def _():
        f = feat_ref[...].astype(jnp.bfloat16)
        we = wexp_ref[...].astype(jnp.bfloat16)
        eh = lax.dot_general(f, we, (((1,), (1,)), ((), ())),
                             preferred_element_type=jnp.float32)
        eh = (eh + bexp_ref[...]).astype(jnp.bfloat16)
        for e in range(E):
            eh_s[:, e * B:(e + 1) * B] = jnp.transpose(
                eh[:, e * L:(e + 1) * L])

        wp = wpri_ref[...].astype(jnp.bfloat16)
        pr = lax.dot_general(wp, f, (((1,), (1,)), ((), ())),
                             preferred_element_type=jnp.float32)
        pr = pr + bpri_ref[...]
        pe = jnp.exp(pr)
        pri_s[...] = pe / jnp.sum(pe, axis=0, keepdims=True)

        s_s[...] = jnp.zeros_like(s_s)

        tc = tcol_ref[...]
        q = tc >> 7
        r = tc & 127
        iot = lax.broadcasted_iota(jnp.int32, (1, VQ), 1)
        oh = jnp.where(q == iot, 1.0, 0.0).astype(jnp.bfloat16)
        gg = lax.dot_general(oh, dbp_ref[...].astype(jnp.bfloat16),
                             (((1,), (0,)), ((), ())),
                             preferred_element_type=jnp.float32)
        rat_s[...] = 1.0 - jnp.take_along_axis(gg[:, 0:128], r, axis=1)
        blv_s[...] = jnp.take_along_axis(gg[:, 128:256], r, axis=1)

    ehb = eh_s[...]
    bl = bl_ref[...].astype(jnp.bfloat16)
    CW = 256

    def chunk(c, masked):
        wc = wl_ref[c * CW:(c + 1) * CW, :].astype(jnp.bfloat16)
        lgc = lax.dot_general(ehb, wc, (((0,), (1,)), ((), ())),
                              preferred_element_type=jnp.float32
                              ).astype(jnp.bfloat16)
        y = lgc + bl[:, c * CW:(c + 1) * CW]
        if masked:
            vidx = (g * VT + c * CW +
                    lax.broadcasted_iota(jnp.int32, (1, CW), 1))
            y = jnp.where(vidx < V, y, jnp.bfloat16(-1e30))
        x = jnp.exp(y)
        ps = x[:, 0:128]
        for j in range(1, CW // 128):
            ps = ps + x[:, j * 128:(j + 1) * 128]
        return ps.astype(jnp.float32)

    def accum(masked, nch):
        ps = [chunk(c, masked) for c in range(nch)]
        while len(ps) > 1:
            ps = [a + b for a, b in zip(ps[::2], ps[1::2])] + (
                [ps[-1]] if len(ps) % 2 else [])
        s_s[...] += ps[0]

    @pl.when(g < NT - 1)
    def _():
        accum(False, VT // 256)

    @pl.when(g == NT - 1)
    def _():
        accum(True, NMC)

        pltpu.make_async_copy(wg_s, wg_s, sem).wait()

        st = jnp.transpose(s_s[...])
        s_row = jnp.sum(st, axis=0, keepdims=True)
        wrows = [(pri_s[e:e + 1, :] /
                  s_row[:, e * B:(e + 1) * B]).astype(jnp.bfloat16)
                 for e in range(E)]
        cbm = lax.broadcasted_iota(jnp.int32, (1, B), 1)

        RC = BT // 4
        pml = []
        for c in range(4):
            wgc = wg_s[c * RC:(c + 1) * RC, :].astype(jnp.bfloat16)
            tlc = lax.dot_general(wgc, ehb, (((1,), (0,)), ((), ())),
                                  preferred_element_type=jnp.float32
                                  ).astype(jnp.bfloat16)
            accc = jnp.zeros((RC, 128), jnp.float32)
            for e in range(E):
                xe = jnp.exp(tlc[:, e * B:(e + 1) * B]) * wrows[e]
                accc = accc + xe.astype(jnp.float32)
            rbm = (c * RC +
                   lax.broadcasted_iota(jnp.int32, (RC, 1), 0)) >> 4
            pml.append(jnp.sum(jnp.where(rbm == cbm, accc, 0.0),
                               axis=-1, keepdims=True))
        pm = jnp.concatenate(pml, axis=0)

        lp = jnp.log(pm) + blv_s[...]
        ratio = rat_s[...]
        num = (-lp) * ratio
        nums = jnp.sum(num.reshape(B, T, 1), axis=1)
        dens = jnp.sum(ratio.reshape(B, T, 1), axis=1)
        ps = nums / dens
        o_ref[...] = (jnp.sum(ps) / (B + 1e-5)).reshape(1, 1)


def kernel(features, W_exp, b_exp, W_lab, b_lab, W_pri, b_pri,
           discard_probs, targets):
    bexp2 = b_exp.reshape(1, E * L)
    bpri_c = b_pri.reshape(E, 1)
    bl2 = b_lab.reshape(1, V)

    npad = VQ * 128 - V
    dp_p = jnp.pad(discard_probs, (0, npad)).reshape(VQ, 128)
    bl_p = jnp.pad(b_lab, (0, npad)).reshape(VQ, 128)
    dbp = jnp.concatenate([dp_p, bl_p], axis=1)

    tflat = targets.astype(jnp.int32).reshape(BT)
    tcol = targets.astype(jnp.int32).reshape(BT, 1)

    loss = pl.pallas_call(
        _mos_body,
        grid_spec=pltpu.PrefetchScalarGridSpec(
            num_scalar_prefetch=1,
            grid=(NT,),
            in_specs=[
                pl.BlockSpec((B, FEAT), lambda i, s: (0, 0)),
                pl.BlockSpec((E * L, FEAT), lambda i, s: (0, 0)),
                pl.BlockSpec((1, E * L), lambda i, s: (0, 0)),
                pl.BlockSpec((E, FEAT), lambda i, s: (0, 0)),
                pl.BlockSpec((E, 1), lambda i, s: (0, 0)),
                pl.BlockSpec((VT, L), lambda i, s: (i, 0)),
                pl.BlockSpec((1, VT), lambda i, s: (0, i)),
                pl.BlockSpec(memory_space=pl.ANY),
                pl.BlockSpec((BT, 1), lambda i, s: (0, 0)),
                pl.BlockSpec((VQ, 256), lambda i, s: (0, 0)),
            ],
            out_specs=pl.BlockSpec((1, 1), lambda i, s: (0, 0)),
            scratch_shapes=[
                pltpu.VMEM((L, BE), jnp.bfloat16),
                pltpu.VMEM((E, B), jnp.float32),
                pltpu.VMEM((BE, 128), jnp.float32),
                pltpu.VMEM((BT, L), jnp.float32),
                pltpu.VMEM((BT, 1), jnp.float32),
                pltpu.VMEM((BT, 1), jnp.float32),
                pltpu.SemaphoreType.DMA,
            ],
        ),
        out_shape=jax.ShapeDtypeStruct((1, 1), jnp.float32),
        compiler_params=pltpu.CompilerParams(
            dimension_semantics=("arbitrary",),
            vmem_limit_bytes=56 * 1024 * 1024,
            disable_bounds_checks=True),
        name="mos_loss_fused",
    )(tflat, features, W_exp, bexp2, W_pri, bpri_c, W_lab, bl2, W_lab,
      tcol, dbp)

    return loss.reshape(1)
